# baseline (speedup 1.0000x reference)
import jax
import jax.numpy as jnp
from jax.experimental import pallas as pl
from jax.experimental.pallas import tpu as pltpu

_K = 16
_N = 256
_BIG = 1e9


def _ctable_kernel(d_ref, c_ref, call_ref, cprev_ref):
    kk = pl.program_id(1)
    d = d_ref[0]
    nn = jax.lax.broadcasted_iota(jnp.int32, (_N, _N), 0)
    ii = jax.lax.broadcasted_iota(jnp.int32, (_N, _N), 1)
    ik = jax.lax.broadcasted_iota(jnp.int32, (_N, _K), 1)

    @pl.when(kk == 0)
    def _init():
        col0 = d[:, _N - 1:_N]
        cprev_ref[:, :] = col0
        c_ref[0] = jnp.where(ik == 0, col0, 0.0)
        call_ref[0] = jnp.where(ii == _N - 1, 1.0, -1.0)

    @pl.when(kk > 0)
    def _step():
        cprev = cprev_ref[:, :]
        shift_mat = (nn == ii + 1).astype(jnp.float32)
        row_shift = jax.lax.dot_general(
            cprev, shift_mat, (((0,), (0,)), ((), ())),
            preferred_element_type=jnp.float32)
        a = d + row_shift
        valid = (ii >= nn) & (ii < _N - kk)
        a_safe = jnp.where(valid, a, _BIG)
        cmin = jnp.min(a_safe, axis=1, keepdims=True)
        nn_col = jax.lax.broadcasted_iota(jnp.int32, (_N, 1), 0)
        newcol = jnp.where(nn_col < _N - kk, cmin, 0.0)
        cprev_ref[:, :] = newcol
        c_ref[0] = jnp.where(ik == kk, newcol, c_ref[0])
        w = jnp.exp(cmin - a_safe)
        s = jnp.sum(w, axis=1, keepdims=True)
        call_ref[0] = jnp.where(valid, w / s, -1.0)


def kernel(input_D_sum):
    b = input_D_sum.shape[0]
    c, call_flat = pl.pallas_call(
        _ctable_kernel,
        grid=(b, _K),
        in_specs=[pl.BlockSpec((1, _N, _N), lambda bi, kk: (bi, 0, 0))],
        out_specs=[
            pl.BlockSpec((1, _N, _K), lambda bi, kk: (bi, 0, 0)),
            pl.BlockSpec((1, _N, _N), lambda bi, kk: (bi, 0, kk)),
        ],
        out_shape=[
            jax.ShapeDtypeStruct((b, _N, _K), jnp.float32),
            jax.ShapeDtypeStruct((b, _N, _K * _N), jnp.float32),
        ],
        scratch_shapes=[pltpu.VMEM((_N, 1), jnp.float32)],
    )(input_D_sum)
    return c, call_flat.reshape(b, _N, _K, _N)


# speedup vs baseline: 1.0967x; 1.0967x over previous
import jax
import jax.numpy as jnp
from jax.experimental import pallas as pl
from jax.experimental.pallas import tpu as pltpu

_K = 16
_N = 256
_BIG = 1e9


def _ctable_kernel(d_ref, c_ref, call_ref, row_ref):
    kk = pl.program_id(1)
    d = d_ref[0]
    nn = jax.lax.broadcasted_iota(jnp.int32, (_N, _N), 0)
    ii = jax.lax.broadcasted_iota(jnp.int32, (_N, _N), 1)
    ik = jax.lax.broadcasted_iota(jnp.int32, (_N, _K), 1)

    def col_to_shifted_row(col):
        return jnp.sum(jnp.where(nn == ii + 1, col, 0.0), axis=0,
                       keepdims=True)

    @pl.when(kk == 0)
    def _init():
        col0 = d[:, _N - 1:_N]
        c_ref[0] = jnp.where(ik == 0, col0, 0.0)
        call_ref[0] = jnp.where(ii == _N - 1, 1.0, -1.0)
        row_ref[:, :] = col_to_shifted_row(col0)

    @pl.when(kk > 0)
    def _step():
        row_shift = row_ref[:, :]
        a = d + row_shift
        valid = (ii >= nn) & (ii < _N - kk)
        a_safe = jnp.where(valid, a, _BIG)
        cmin = jnp.min(a_safe, axis=1, keepdims=True)
        nn_col = jax.lax.broadcasted_iota(jnp.int32, (_N, 1), 0)
        newcol = jnp.where(nn_col < _N - kk, cmin, 0.0)
        c_ref[0] = jnp.where(ik == kk, newcol, c_ref[0])
        row_ref[:, :] = col_to_shifted_row(newcol)
        w = jnp.exp(cmin - a_safe)
        s = jnp.sum(w, axis=1, keepdims=True)
        call_ref[0] = jnp.where(valid, w / s, -1.0)


def kernel(input_D_sum):
    b = input_D_sum.shape[0]
    c, call_flat = pl.pallas_call(
        _ctable_kernel,
        grid=(b, _K),
        in_specs=[pl.BlockSpec((1, _N, _N), lambda bi, kk: (bi, 0, 0))],
        out_specs=[
            pl.BlockSpec((1, _N, _K), lambda bi, kk: (bi, 0, 0)),
            pl.BlockSpec((1, _N, _N), lambda bi, kk: (bi, 0, kk)),
        ],
        out_shape=[
            jax.ShapeDtypeStruct((b, _N, _K), jnp.float32),
            jax.ShapeDtypeStruct((b, _N, _K * _N), jnp.float32),
        ],
        scratch_shapes=[pltpu.VMEM((1, _N), jnp.float32)],
    )(input_D_sum)
    return c, call_flat.reshape(b, _N, _K, _N)
